# Initial kernel scaffold; baseline (speedup 1.0000x reference)
import jax
import jax.numpy as jnp
from jax.experimental import pallas as pl


def kernel(x, edge_index, curr_idx, dest_idx, neighbor_indices, edge_attr, lin_e1_W, lin_e1_b, mlp1_W1, mlp1_b1, mlp1_W2, mlp1_b2, lin_e2_W, lin_e2_b, mlp2_W1, mlp2_b1, mlp2_W2, mlp2_b2, head_W1, head_b1, head_W2, head_b2):
    raise NotImplementedError("write your pallas kernel here")


# speedup vs baseline: 2.1894x; 2.1894x over previous
import functools

import jax
import jax.numpy as jnp
from jax import lax
from jax.experimental import pallas as pl
from jax.experimental.pallas import tpu as pltpu
from jax.experimental.pallas import tpu_sc as plsc

N = 10000
E = 320000
H = 128
ED = 16
K = 32

NC = 2
NS = 16
NW = NC * NS
CHUNK = 128
NCHUNK = -(-E // (NW * CHUNK))
E_PAD = NW * CHUNK * NCHUNK
EPT = E_PAD // NW
AGGR_ROWS = 10240
ZROWS = AGGR_ROWS // NS
RROWS = 640
RLAST = N - 15 * RROWS
LANES = 16


def _sc_body(h_hbm, e_hbm, src_hbm, dst_hbm, out_hbm,
             idx_s, idx_d, xbuf, ebuf, sem, aggr):
    cid = lax.axis_index("c")
    sid = lax.axis_index("s")
    wid = cid * NS + sid

    def _zrow(r, _):
        for g in range(H // LANES):
            xbuf[r, pl.ds(g * LANES, LANES)] = jnp.zeros((LANES,), jnp.float32)
        return 0
    lax.fori_loop(0, CHUNK, _zrow, 0)
    zbase = sid * ZROWS
    for i in range(ZROWS // CHUNK):
        pltpu.sync_copy(xbuf, aggr.at[pl.ds(zbase + i * CHUNK, CHUNK)])
    plsc.subcore_barrier()

    ebase = wid * EPT

    def _chunk(ci, _):
        base = ebase + ci * CHUNK
        pltpu.sync_copy(src_hbm.at[pl.ds(base, CHUNK)], idx_s)
        pltpu.sync_copy(dst_hbm.at[pl.ds(base, CHUNK)], idx_d)
        pltpu.async_copy(h_hbm.at[idx_s], xbuf, sem).wait()
        pltpu.sync_copy(e_hbm.at[pl.ds(base, CHUNK)], ebuf)

        def _row(r, _):
            for g in range(H // LANES):
                sl = pl.ds(g * LANES, LANES)
                xbuf[r, sl] = jnp.maximum(xbuf[r, sl] + ebuf[r, sl], 0.0)
            return 0
        lax.fori_loop(0, CHUNK, _row, 0)
        pltpu.sync_copy(xbuf, aggr.at[idx_d], add=True)
        return 0

    lax.fori_loop(0, NCHUNK, _chunk, 0)
    plsc.subcore_barrier()
    rbase = sid * RROWS

    @pl.when(sid < NS - 1)
    def _():
        pltpu.sync_copy(aggr.at[pl.ds(rbase, RROWS)],
                        out_hbm.at[cid, pl.ds(rbase, RROWS)])

    @pl.when(sid == NS - 1)
    def _():
        pltpu.sync_copy(aggr.at[pl.ds((NS - 1) * RROWS, RLAST)],
                        out_hbm.at[cid, pl.ds((NS - 1) * RROWS, RLAST)])


_sc_layer = pl.kernel(
    _sc_body,
    out_type=jax.ShapeDtypeStruct((NC, N, H), jnp.float32),
    mesh=plsc.VectorSubcoreMesh(core_axis_name="c", subcore_axis_name="s"),
    scratch_types=[
        pltpu.VMEM((CHUNK,), jnp.int32),
        pltpu.VMEM((CHUNK,), jnp.int32),
        pltpu.VMEM((CHUNK, H), jnp.float32),
        pltpu.VMEM((CHUNK, H), jnp.float32),
        pltpu.SemaphoreType.DMA,
        pltpu.VMEM_SHARED((AGGR_ROWS, H), jnp.float32),
    ],
)


def _edge_lin_body(ea_ref, w_ref, b_ref, o_ref):
    o_ref[...] = (jnp.dot(ea_ref[...], w_ref[...],
                          preferred_element_type=jnp.float32) + b_ref[...])


def _edge_lin(ea, w_t, b):
    be = 4096
    return pl.pallas_call(
        _edge_lin_body,
        grid=(E_PAD // be,),
        in_specs=[
            pl.BlockSpec((be, ED), lambda i: (i, 0)),
            pl.BlockSpec((ED, H), lambda i: (0, 0)),
            pl.BlockSpec((1, H), lambda i: (0, 0)),
        ],
        out_specs=pl.BlockSpec((be, H), lambda i: (i, 0)),
        out_shape=jax.ShapeDtypeStruct((E_PAD, H), jnp.float32),
    )(ea, w_t, b.reshape(1, H))


def _mlp_body(x_ref, p_ref, w1_ref, b1_ref, w2_ref, b2_ref, o_ref, *, final_relu):
    z = x_ref[...] + p_ref[0] + p_ref[1]
    hh = jnp.maximum(jnp.dot(z, w1_ref[...],
                             preferred_element_type=jnp.float32) + b1_ref[...], 0.0)
    out = jnp.dot(hh, w2_ref[...], preferred_element_type=jnp.float32) + b2_ref[...]
    if final_relu:
        out = jnp.maximum(out, 0.0)
    o_ref[...] = out


def _mlp(x, partials, w1_t, b1, w2_t, b2, final_relu):
    bn = 2000
    return pl.pallas_call(
        functools.partial(_mlp_body, final_relu=final_relu),
        grid=(N // bn,),
        in_specs=[
            pl.BlockSpec((bn, H), lambda i: (i, 0)),
            pl.BlockSpec((NC, bn, H), lambda i: (0, i, 0)),
            pl.BlockSpec((H, H), lambda i: (0, 0)),
            pl.BlockSpec((1, H), lambda i: (0, 0)),
            pl.BlockSpec((H, H), lambda i: (0, 0)),
            pl.BlockSpec((1, H), lambda i: (0, 0)),
        ],
        out_specs=pl.BlockSpec((bn, H), lambda i: (i, 0)),
        out_shape=jax.ShapeDtypeStruct((N, H), jnp.float32),
    )(x, partials, w1_t, b1.reshape(1, H), w2_t, b2.reshape(1, H))


def _head_body(cat_ref, w1_ref, b1_ref, w2_ref, b2_ref, o_ref):
    hh = jnp.maximum(jnp.dot(cat_ref[...], w1_ref[...],
                             preferred_element_type=jnp.float32) + b1_ref[...], 0.0)
    o_ref[...] = (jnp.dot(hh, w2_ref[...],
                          preferred_element_type=jnp.float32) + b2_ref[...])


def _head(cat, w1_t, b1, w2_t, b2):
    return pl.pallas_call(
        _head_body,
        out_shape=jax.ShapeDtypeStruct((K, 1), jnp.float32),
    )(cat, w1_t, b1.reshape(1, H), w2_t, b2.reshape(1, 1))


def kernel(x, edge_index, curr_idx, dest_idx, neighbor_indices, edge_attr,
           lin_e1_W, lin_e1_b, mlp1_W1, mlp1_b1, mlp1_W2, mlp1_b2,
           lin_e2_W, lin_e2_b, mlp2_W1, mlp2_b1, mlp2_W2, mlp2_b2,
           head_W1, head_b1, head_W2, head_b2):
    pad = E_PAD - E
    src = jnp.concatenate([edge_index[0], jnp.zeros((pad,), jnp.int32)])
    dump = N + (jnp.arange(pad, dtype=jnp.int32) % (AGGR_ROWS - N))
    dst = jnp.concatenate([edge_index[1], dump])
    ea = jnp.concatenate([edge_attr, jnp.zeros((pad, ED), jnp.float32)])

    e1 = _edge_lin(ea, lin_e1_W.T, lin_e1_b)
    p1 = _sc_layer(x, e1, src, dst)
    h1 = _mlp(x, p1, mlp1_W1.T, mlp1_b1, mlp1_W2.T, mlp1_b2, final_relu=True)

    e2 = _edge_lin(ea, lin_e2_W.T, lin_e2_b)
    p2 = _sc_layer(h1, e2, src, dst)
    h2 = _mlp(h1, p2, mlp2_W1.T, mlp2_b1, mlp2_W2.T, mlp2_b2, final_relu=False)

    curr = h2[curr_idx]
    dest = h2[dest_idx]
    nbr = h2[neighbor_indices]
    cat = jnp.concatenate([
        jnp.broadcast_to(curr, (K, H)),
        jnp.broadcast_to(dest, (K, H)),
        nbr,
    ], axis=1)
    q = _head(cat, head_W1.T, head_b1, head_W2.T, head_b2)
    return q[:, 0]


# speedup vs baseline: 4.7197x; 2.1557x over previous
import functools

import jax
import jax.numpy as jnp
from jax import lax
from jax.experimental import pallas as pl
from jax.experimental.pallas import tpu as pltpu
from jax.experimental.pallas import tpu_sc as plsc

N = 10000
E = 320000
H = 128
ED = 16
K = 32

NC = 2
NS = 16
NW = NC * NS
CHUNK = 80
NCHUNK = E // (NW * CHUNK)
EPT = E // NW
AGGR_ROWS = 10240
ZROWS = AGGR_ROWS // NS
RROWS = 640
RLAST = N - 15 * RROWS
LANES = 16


def _sc_body(h_hbm, e_hbm, src_hbm, dst_hbm, out_hbm,
             srcb, dstb, xbuf, ebuf,
             gsem0, gsem1, esem0, esem1, ssem0, ssem1,
             isem0, isem1, isem2, isem3, aggr):
    cid = lax.axis_index("c")
    sid = lax.axis_index("s")
    wid = cid * NS + sid
    gsem = (gsem0, gsem1)
    esem = (esem0, esem1)
    ssem = (ssem0, ssem1)
    isem = (isem0, isem1, isem2, isem3)

    @plsc.parallel_loop(0, CHUNK, unroll=2)
    def _(r):
        for g in range(H // LANES):
            xbuf[0, r, pl.ds(g * LANES, LANES)] = jnp.zeros((LANES,), jnp.float32)
    zbase = sid * ZROWS
    for i in range(ZROWS // CHUNK):
        pltpu.sync_copy(xbuf.at[0], aggr.at[pl.ds(zbase + i * CHUNK, CHUNK)])
    plsc.subcore_barrier()

    ebase = wid * EPT

    def _issue_idx(ci, bb, bd):
        base = ebase + ci * CHUNK
        pltpu.async_copy(src_hbm.at[pl.ds(base, CHUNK)], srcb.at[bb], isem[bd])
        pltpu.async_copy(dst_hbm.at[pl.ds(base, CHUNK)], dstb.at[bd], isem[bd])

    def _wait_idx(ci, bb, bd):
        base = ebase + ci * CHUNK
        pltpu.make_async_copy(src_hbm.at[pl.ds(base, CHUNK)], srcb.at[bb],
                              isem[bd]).wait()
        pltpu.make_async_copy(dst_hbm.at[pl.ds(base, CHUNK)], dstb.at[bd],
                              isem[bd]).wait()

    def _issue_loads(ci, bb):
        pltpu.async_copy(h_hbm.at[srcb.at[bb]], xbuf.at[bb], gsem[bb])
        pltpu.async_copy(e_hbm.at[pl.ds(ebase + ci * CHUNK, CHUNK)],
                         ebuf.at[bb], esem[bb])

    def _wait_loads(ci, bb):
        pltpu.make_async_copy(h_hbm.at[srcb.at[bb]], xbuf.at[bb],
                              gsem[bb]).wait()
        pltpu.make_async_copy(e_hbm.at[pl.ds(ebase + ci * CHUNK, CHUNK)],
                              ebuf.at[bb], esem[bb]).wait()

    def _issue_scat(bb, bd):
        pltpu.async_copy(xbuf.at[bb], aggr.at[dstb.at[bd]], ssem[bb], add=True)

    def _wait_scat(bb, bd):
        pltpu.make_async_copy(xbuf.at[bb], aggr.at[dstb.at[bd]],
                              ssem[bb]).wait()

    def _compute(bb):
        @plsc.parallel_loop(0, CHUNK, unroll=2)
        def _(r):
            for g in range(H // LANES):
                sl = pl.ds(g * LANES, LANES)
                xbuf[bb, r, sl] = jnp.maximum(xbuf[bb, r, sl] + ebuf[bb, r, sl],
                                              0.0)

    def _step(ci, k, first=False, last=False):
        bb, bd = k % 2, k
        bbn, bdn = (k + 1) % 2, (k + 1) % 4
        if not first:
            _wait_scat(bbn, (k + 3) % 4)
        if not last:
            _wait_idx(ci + 1, bbn, bdn)
            _issue_loads(ci + 1, bbn)
        _wait_loads(ci, bb)
        if not last:
            @pl.when(ci + 2 < NCHUNK)
            def _():
                _issue_idx(ci + 2, bb, (k + 2) % 4)
        _compute(bb)
        _issue_scat(bb, bd)

    _issue_idx(0, 0, 0)
    _issue_idx(1, 1, 1)
    _wait_idx(0, 0, 0)
    _issue_loads(0, 0)

    def _outer(it, _):
        ci0 = it * 4
        _step(ci0, 0, first=False)
        _step(ci0 + 1, 1)
        _step(ci0 + 2, 2)
        _step(ci0 + 3, 3)
        return 0

    _step(0, 0, first=True)
    _step(1, 1)
    _step(2, 2)
    _step(3, 3)
    lax.fori_loop(1, (NCHUNK - 1) // 4, _outer, 0)
    _step(NCHUNK - 1, (NCHUNK - 1) % 4, last=True)
    _wait_scat((NCHUNK - 1) % 2, (NCHUNK - 1) % 4)
    plsc.subcore_barrier()
    rbase = sid * RROWS

    @pl.when(sid < NS - 1)
    def _():
        pltpu.sync_copy(aggr.at[pl.ds(rbase, RROWS)],
                        out_hbm.at[cid, pl.ds(rbase, RROWS)])

    @pl.when(sid == NS - 1)
    def _():
        pltpu.sync_copy(aggr.at[pl.ds((NS - 1) * RROWS, RLAST)],
                        out_hbm.at[cid, pl.ds((NS - 1) * RROWS, RLAST)])


_sc_layer = pl.kernel(
    _sc_body,
    out_type=jax.ShapeDtypeStruct((NC, N, H), jnp.float32),
    mesh=plsc.VectorSubcoreMesh(core_axis_name="c", subcore_axis_name="s"),
    scratch_types=[
        pltpu.VMEM((2, CHUNK), jnp.int32),
        pltpu.VMEM((4, CHUNK), jnp.int32),
        pltpu.VMEM((2, CHUNK, H), jnp.float32),
        pltpu.VMEM((2, CHUNK, H), jnp.float32),
        pltpu.SemaphoreType.DMA,
        pltpu.SemaphoreType.DMA,
        pltpu.SemaphoreType.DMA,
        pltpu.SemaphoreType.DMA,
        pltpu.SemaphoreType.DMA,
        pltpu.SemaphoreType.DMA,
        pltpu.SemaphoreType.DMA,
        pltpu.SemaphoreType.DMA,
        pltpu.SemaphoreType.DMA,
        pltpu.SemaphoreType.DMA,
        pltpu.VMEM_SHARED((AGGR_ROWS, H), jnp.float32),
    ],
)


def _edge_lin_body(ea_ref, w_ref, b_ref, o_ref):
    o_ref[...] = (jnp.dot(ea_ref[...], w_ref[...],
                          preferred_element_type=jnp.float32) + b_ref[...])


def _edge_lin(ea, w_t, b):
    be = 4000
    return pl.pallas_call(
        _edge_lin_body,
        grid=(E // be,),
        in_specs=[
            pl.BlockSpec((be, ED), lambda i: (i, 0)),
            pl.BlockSpec((ED, H), lambda i: (0, 0)),
            pl.BlockSpec((1, H), lambda i: (0, 0)),
        ],
        out_specs=pl.BlockSpec((be, H), lambda i: (i, 0)),
        out_shape=jax.ShapeDtypeStruct((E, H), jnp.float32),
    )(ea, w_t, b.reshape(1, H))


def _mlp_body(x_ref, p_ref, w1_ref, b1_ref, w2_ref, b2_ref, o_ref, *, final_relu):
    z = x_ref[...] + p_ref[0] + p_ref[1]
    hh = jnp.maximum(jnp.dot(z, w1_ref[...],
                             preferred_element_type=jnp.float32) + b1_ref[...], 0.0)
    out = jnp.dot(hh, w2_ref[...], preferred_element_type=jnp.float32) + b2_ref[...]
    if final_relu:
        out = jnp.maximum(out, 0.0)
    o_ref[...] = out


def _mlp(x, partials, w1_t, b1, w2_t, b2, final_relu):
    bn = 2000
    return pl.pallas_call(
        functools.partial(_mlp_body, final_relu=final_relu),
        grid=(N // bn,),
        in_specs=[
            pl.BlockSpec((bn, H), lambda i: (i, 0)),
            pl.BlockSpec((NC, bn, H), lambda i: (0, i, 0)),
            pl.BlockSpec((H, H), lambda i: (0, 0)),
            pl.BlockSpec((1, H), lambda i: (0, 0)),
            pl.BlockSpec((H, H), lambda i: (0, 0)),
            pl.BlockSpec((1, H), lambda i: (0, 0)),
        ],
        out_specs=pl.BlockSpec((bn, H), lambda i: (i, 0)),
        out_shape=jax.ShapeDtypeStruct((N, H), jnp.float32),
    )(x, partials, w1_t, b1.reshape(1, H), w2_t, b2.reshape(1, H))


def _head_body(cat_ref, w1_ref, b1_ref, w2_ref, b2_ref, o_ref):
    hh = jnp.maximum(jnp.dot(cat_ref[...], w1_ref[...],
                             preferred_element_type=jnp.float32) + b1_ref[...], 0.0)
    o_ref[...] = (jnp.dot(hh, w2_ref[...],
                          preferred_element_type=jnp.float32) + b2_ref[...])


def _head(cat, w1_t, b1, w2_t, b2):
    return pl.pallas_call(
        _head_body,
        out_shape=jax.ShapeDtypeStruct((K, 1), jnp.float32),
    )(cat, w1_t, b1.reshape(1, H), w2_t, b2.reshape(1, 1))


def kernel(x, edge_index, curr_idx, dest_idx, neighbor_indices, edge_attr,
           lin_e1_W, lin_e1_b, mlp1_W1, mlp1_b1, mlp1_W2, mlp1_b2,
           lin_e2_W, lin_e2_b, mlp2_W1, mlp2_b1, mlp2_W2, mlp2_b2,
           head_W1, head_b1, head_W2, head_b2):
    src = edge_index[0]
    dst = edge_index[1]

    e1 = _edge_lin(edge_attr, lin_e1_W.T, lin_e1_b)
    p1 = _sc_layer(x, e1, src, dst)
    h1 = _mlp(x, p1, mlp1_W1.T, mlp1_b1, mlp1_W2.T, mlp1_b2, final_relu=True)

    e2 = _edge_lin(edge_attr, lin_e2_W.T, lin_e2_b)
    p2 = _sc_layer(h1, e2, src, dst)
    h2 = _mlp(h1, p2, mlp2_W1.T, mlp2_b1, mlp2_W2.T, mlp2_b2, final_relu=False)

    curr = h2[curr_idx]
    dest = h2[dest_idx]
    nbr = h2[neighbor_indices]
    cat = jnp.concatenate([
        jnp.broadcast_to(curr, (K, H)),
        jnp.broadcast_to(dest, (K, H)),
        nbr,
    ], axis=1)
    q = _head(cat, head_W1.T, head_b1, head_W2.T, head_b2)
    return q[:, 0]


# speedup vs baseline: 5.1897x; 1.0996x over previous
import functools

import jax
import jax.numpy as jnp
import numpy as np
from jax import lax
from jax.experimental import pallas as pl
from jax.experimental.pallas import tpu as pltpu
from jax.experimental.pallas import tpu_sc as plsc

N = 10000
E = 320000
H = 128
ED = 16
K = 32

NC = 2
NS = 16
NW = NC * NS
CHUNK = 80
NCHUNK = E // (NW * CHUNK)
EPT = E // NW
AGGR_ROWS = 10240
ZROWS = AGGR_ROWS // NS
RROWS = 640
RLAST = N - 15 * RROWS
LANES = 16


_MSK = np.int32(-65536)


def _lo(w):
    return lax.bitcast_convert_type(jnp.left_shift(w, 16), jnp.float32)


def _hi(w):
    return lax.bitcast_convert_type(jnp.bitwise_and(w, _MSK), jnp.float32)


def _sc_body(h_hbm, e_hbm, src_hbm, dst_hbm, out_hbm,
             srcb, dstb, xbuf, ebuf,
             gsem0, gsem1, esem0, esem1, ssem0, ssem1,
             isem0, isem1, isem2, isem3, aggr):
    cid = lax.axis_index("c")
    sid = lax.axis_index("s")
    wid = cid * NS + sid
    gsem = (gsem0, gsem1)
    esem = (esem0, esem1)
    ssem = (ssem0, ssem1)
    isem = (isem0, isem1, isem2, isem3)

    @plsc.parallel_loop(0, CHUNK, unroll=2)
    def _(r):
        for g in range(H // LANES):
            xbuf[0, r, pl.ds(g * LANES, LANES)] = jnp.zeros((LANES,), jnp.float32)
    zbase = sid * ZROWS
    for i in range(ZROWS // CHUNK):
        pltpu.sync_copy(xbuf.at[0], aggr.at[pl.ds(zbase + i * CHUNK, CHUNK)])
    plsc.subcore_barrier()

    ebase = wid * EPT
    pbase = wid * (EPT // 2)

    def _issue_idx(ci, bb, bd):
        base = ebase + ci * CHUNK
        pltpu.async_copy(src_hbm.at[pl.ds(base, CHUNK)], srcb.at[bb], isem[bd])
        pltpu.async_copy(dst_hbm.at[pl.ds(base, CHUNK)], dstb.at[bd], isem[bd])

    def _wait_idx(ci, bb, bd):
        base = ebase + ci * CHUNK
        pltpu.make_async_copy(src_hbm.at[pl.ds(base, CHUNK)], srcb.at[bb],
                              isem[bd]).wait()
        pltpu.make_async_copy(dst_hbm.at[pl.ds(base, CHUNK)], dstb.at[bd],
                              isem[bd]).wait()

    def _issue_loads(ci, bb):
        pltpu.async_copy(h_hbm.at[srcb.at[bb]], xbuf.at[bb], gsem[bb])
        pltpu.async_copy(e_hbm.at[pl.ds(pbase + ci * (CHUNK // 2), CHUNK // 2)],
                         ebuf.at[bb], esem[bb])

    def _wait_loads(ci, bb):
        pltpu.make_async_copy(h_hbm.at[srcb.at[bb]], xbuf.at[bb],
                              gsem[bb]).wait()
        pltpu.make_async_copy(e_hbm.at[pl.ds(pbase + ci * (CHUNK // 2), CHUNK // 2)],
                              ebuf.at[bb], esem[bb]).wait()

    def _issue_scat(bb, bd):
        pltpu.async_copy(xbuf.at[bb], aggr.at[dstb.at[bd]], ssem[bb], add=True)

    def _wait_scat(bb, bd):
        pltpu.make_async_copy(xbuf.at[bb], aggr.at[dstb.at[bd]],
                              ssem[bb]).wait()

    def _compute(bb):
        @plsc.parallel_loop(0, CHUNK // 2, unroll=2)
        def _(rp):
            r0 = 2 * rp
            for g in range(H // LANES):
                sl = pl.ds(LANES * g, LANES)
                we = ebuf[bb, rp, sl]
                xbuf[bb, r0, sl] = jnp.maximum(xbuf[bb, r0, sl] + _lo(we), 0.0)
                xbuf[bb, r0 + 1, sl] = jnp.maximum(
                    xbuf[bb, r0 + 1, sl] + _hi(we), 0.0)

    def _step(ci, k, first=False, last=False):
        bb, bd = k % 2, k
        bbn, bdn = (k + 1) % 2, (k + 1) % 4
        if not first:
            _wait_scat(bbn, (k + 3) % 4)
        if not last:
            _wait_idx(ci + 1, bbn, bdn)
            _issue_loads(ci + 1, bbn)
        _wait_loads(ci, bb)
        if not last:
            @pl.when(ci + 2 < NCHUNK)
            def _():
                _issue_idx(ci + 2, bb, (k + 2) % 4)
        _compute(bb)
        _issue_scat(bb, bd)

    _issue_idx(0, 0, 0)
    _issue_idx(1, 1, 1)
    _wait_idx(0, 0, 0)
    _issue_loads(0, 0)

    def _outer(it, _):
        ci0 = it * 4
        _step(ci0, 0, first=False)
        _step(ci0 + 1, 1)
        _step(ci0 + 2, 2)
        _step(ci0 + 3, 3)
        return 0

    _step(0, 0, first=True)
    _step(1, 1)
    _step(2, 2)
    _step(3, 3)
    lax.fori_loop(1, (NCHUNK - 1) // 4, _outer, 0)
    _step(NCHUNK - 1, (NCHUNK - 1) % 4, last=True)
    _wait_scat((NCHUNK - 1) % 2, (NCHUNK - 1) % 4)
    plsc.subcore_barrier()
    rbase = sid * RROWS

    @pl.when(sid < NS - 1)
    def _():
        pltpu.sync_copy(aggr.at[pl.ds(rbase, RROWS)],
                        out_hbm.at[cid, pl.ds(rbase, RROWS)])

    @pl.when(sid == NS - 1)
    def _():
        pltpu.sync_copy(aggr.at[pl.ds((NS - 1) * RROWS, RLAST)],
                        out_hbm.at[cid, pl.ds((NS - 1) * RROWS, RLAST)])


_sc_layer = pl.kernel(
    _sc_body,
    out_type=jax.ShapeDtypeStruct((NC, N, H), jnp.float32),
    mesh=plsc.VectorSubcoreMesh(core_axis_name="c", subcore_axis_name="s"),
    scratch_types=[
        pltpu.VMEM((2, CHUNK), jnp.int32),
        pltpu.VMEM((4, CHUNK), jnp.int32),
        pltpu.VMEM((2, CHUNK, H), jnp.float32),
        pltpu.VMEM((2, CHUNK // 2, H), jnp.int32),
        pltpu.SemaphoreType.DMA,
        pltpu.SemaphoreType.DMA,
        pltpu.SemaphoreType.DMA,
        pltpu.SemaphoreType.DMA,
        pltpu.SemaphoreType.DMA,
        pltpu.SemaphoreType.DMA,
        pltpu.SemaphoreType.DMA,
        pltpu.SemaphoreType.DMA,
        pltpu.SemaphoreType.DMA,
        pltpu.SemaphoreType.DMA,
        pltpu.VMEM_SHARED((AGGR_ROWS, H), jnp.float32),
    ],
)


def _edge_lin_body(ea_ref, w_ref, b_ref, o_ref):
    e2 = (jnp.dot(ea_ref[...], w_ref[...],
                  preferred_element_type=jnp.float32) + b_ref[...])
    u = lax.convert_element_type(
        lax.bitcast_convert_type(e2.astype(jnp.bfloat16), jnp.uint16),
        jnp.int32)
    o_ref[...] = jnp.bitwise_or(u[:, :H], jnp.left_shift(u[:, H:], 16))


def _edge_lin(ea2, w_bd, b2):
    be2 = 2000
    return pl.pallas_call(
        _edge_lin_body,
        grid=(E // 2 // be2,),
        in_specs=[
            pl.BlockSpec((be2, 2 * ED), lambda i: (i, 0)),
            pl.BlockSpec((2 * ED, 2 * H), lambda i: (0, 0)),
            pl.BlockSpec((1, 2 * H), lambda i: (0, 0)),
        ],
        out_specs=pl.BlockSpec((be2, H), lambda i: (i, 0)),
        out_shape=jax.ShapeDtypeStruct((E // 2, H), jnp.int32),
    )(ea2, w_bd, b2.reshape(1, 2 * H))


def _mlp_body(x_ref, p_ref, w1_ref, b1_ref, w2_ref, b2_ref, o_ref, *, final_relu):
    z = x_ref[...] + p_ref[0] + p_ref[1]
    hh = jnp.maximum(jnp.dot(z, w1_ref[...],
                             preferred_element_type=jnp.float32) + b1_ref[...], 0.0)
    out = jnp.dot(hh, w2_ref[...], preferred_element_type=jnp.float32) + b2_ref[...]
    if final_relu:
        out = jnp.maximum(out, 0.0)
    o_ref[...] = out


def _mlp(x, partials, w1_t, b1, w2_t, b2, final_relu):
    bn = 2000
    return pl.pallas_call(
        functools.partial(_mlp_body, final_relu=final_relu),
        grid=(N // bn,),
        in_specs=[
            pl.BlockSpec((bn, H), lambda i: (i, 0)),
            pl.BlockSpec((NC, bn, H), lambda i: (0, i, 0)),
            pl.BlockSpec((H, H), lambda i: (0, 0)),
            pl.BlockSpec((1, H), lambda i: (0, 0)),
            pl.BlockSpec((H, H), lambda i: (0, 0)),
            pl.BlockSpec((1, H), lambda i: (0, 0)),
        ],
        out_specs=pl.BlockSpec((bn, H), lambda i: (i, 0)),
        out_shape=jax.ShapeDtypeStruct((N, H), jnp.float32),
    )(x, partials, w1_t, b1.reshape(1, H), w2_t, b2.reshape(1, H))


def _head_body(cat_ref, w1_ref, b1_ref, w2_ref, b2_ref, o_ref):
    hh = jnp.maximum(jnp.dot(cat_ref[...], w1_ref[...],
                             preferred_element_type=jnp.float32) + b1_ref[...], 0.0)
    o_ref[...] = (jnp.dot(hh, w2_ref[...],
                          preferred_element_type=jnp.float32) + b2_ref[...])


def _head(cat, w1_t, b1, w2_t, b2):
    return pl.pallas_call(
        _head_body,
        out_shape=jax.ShapeDtypeStruct((K, 1), jnp.float32),
    )(cat, w1_t, b1.reshape(1, H), w2_t, b2.reshape(1, 1))


def kernel(x, edge_index, curr_idx, dest_idx, neighbor_indices, edge_attr,
           lin_e1_W, lin_e1_b, mlp1_W1, mlp1_b1, mlp1_W2, mlp1_b2,
           lin_e2_W, lin_e2_b, mlp2_W1, mlp2_b1, mlp2_W2, mlp2_b2,
           head_W1, head_b1, head_W2, head_b2):
    src = edge_index[0]
    dst = edge_index[1]
    ea2 = edge_attr.reshape(E // 2, 2 * ED)

    def _bd(w_t):
        z = jnp.zeros((ED, H), jnp.float32)
        return jnp.concatenate([
            jnp.concatenate([w_t, z], axis=1),
            jnp.concatenate([z, w_t], axis=1),
        ], axis=0)

    e1 = _edge_lin(ea2, _bd(lin_e1_W.T), jnp.concatenate([lin_e1_b, lin_e1_b]))
    p1 = _sc_layer(x, e1, src, dst)
    h1 = _mlp(x, p1, mlp1_W1.T, mlp1_b1, mlp1_W2.T, mlp1_b2, final_relu=True)

    e2 = _edge_lin(ea2, _bd(lin_e2_W.T), jnp.concatenate([lin_e2_b, lin_e2_b]))
    p2 = _sc_layer(h1, e2, src, dst)
    h2 = _mlp(h1, p2, mlp2_W1.T, mlp2_b1, mlp2_W2.T, mlp2_b2, final_relu=False)

    curr = h2[curr_idx]
    dest = h2[dest_idx]
    nbr = h2[neighbor_indices]
    cat = jnp.concatenate([
        jnp.broadcast_to(curr, (K, H)),
        jnp.broadcast_to(dest, (K, H)),
        nbr,
    ], axis=1)
    q = _head(cat, head_W1.T, head_b1, head_W2.T, head_b2)
    return q[:, 0]
